# baseline (speedup 1.0000x reference)
import functools

import jax
import jax.numpy as jnp
from jax import lax
from jax.experimental import pallas as pl
from jax.experimental.pallas import tpu as pltpu
from jax.experimental.pallas import tpu_sc as plsc

N = 10000
D = 128
E = 320000
EPS = 1e-5

NC = 2
NS = 16
NW = NC * NS
K = 64
CPT = 160
E_PAD = NW * CPT * K
PAIRS = CPT // 2 - 1
N_PAD = 10112
ROWS_PT = N_PAD // NS

assert CPT == 2 * PAIRS + 2
assert N_PAD % (8 * NS) == 0


def _sc_agg_body(with_deg, h_hbm, src_hbm, dst_hbm, z128_hbm, agg_out,
                 deg_out, agg_sh, didx_big, sidx0, sidx1, rows0, rows1,
                 sem0, sem1):
    cid = lax.axis_index("c")
    sid = lax.axis_index("s")
    wid = sid * NC + cid
    r0 = sid * ROWS_PT

    pltpu.sync_copy(z128_hbm.at[pl.ds(r0, ROWS_PT)],
                    agg_sh.at[pl.ds(r0, ROWS_PT)])
    pltpu.sync_copy(dst_hbm.at[wid], didx_big)
    plsc.subcore_barrier()

    ebase = wid * CPT * K

    def fire(c, sidx, rows, sem):
        pltpu.sync_copy(src_hbm.at[pl.ds(ebase + c * K, K)], sidx)
        return pltpu.async_copy(h_hbm.at[sidx], rows, sem)

    def drain(desc, c, rows):
        desc.wait()
        pltpu.sync_copy(rows, agg_sh.at[didx_big.at[c]], add=True)

    d0 = fire(0, sidx0, rows0, sem0)

    def pair(jj, carry):
        c = 2 * jj
        d1 = fire(c + 1, sidx1, rows1, sem1)
        drain(d0, c, rows0)
        fire(c + 2, sidx0, rows0, sem0)
        drain(d1, c + 1, rows1)
        return carry

    lax.fori_loop(0, PAIRS, pair, 0)
    d1 = fire(CPT - 1, sidx1, rows1, sem1)
    drain(d0, CPT - 2, rows0)
    drain(d1, CPT - 1, rows1)

    plsc.subcore_barrier()
    pltpu.sync_copy(agg_sh.at[pl.ds(r0, ROWS_PT)],
                    agg_out.at[cid, pl.ds(r0, ROWS_PT)])

    if with_deg:
        plsc.subcore_barrier()
        pltpu.sync_copy(z128_hbm.at[pl.ds(r0, ROWS_PT)],
                        agg_sh.at[pl.ds(r0, ROWS_PT)])
        for i in range(K):
            for j in range(D // 16):
                rows0[i, pl.ds(j * 16, 16)] = jnp.ones((16,), jnp.float32)
        plsc.subcore_barrier()

        def deg_chunk(j, carry):
            pltpu.sync_copy(rows0, agg_sh.at[didx_big.at[j]], add=True)
            return carry

        lax.fori_loop(0, CPT, deg_chunk, 0)
        plsc.subcore_barrier()
        pltpu.sync_copy(agg_sh.at[pl.ds(r0, ROWS_PT)],
                        deg_out.at[cid, pl.ds(r0, ROWS_PT)])


def _make_sc_agg(with_deg):
    mesh = plsc.VectorSubcoreMesh(core_axis_name="c", subcore_axis_name="s")
    out_type = (jax.ShapeDtypeStruct((NC, N_PAD, D), jnp.float32),
                jax.ShapeDtypeStruct((NC, N_PAD, D), jnp.float32))
    scratch = [
        pltpu.VMEM_SHARED((N_PAD, D), jnp.float32),
        pltpu.VMEM((CPT, K), jnp.int32),
        pltpu.VMEM((K,), jnp.int32),
        pltpu.VMEM((K,), jnp.int32),
        pltpu.VMEM((K, D), jnp.float32),
        pltpu.VMEM((K, D), jnp.float32),
        pltpu.SemaphoreType.DMA,
        pltpu.SemaphoreType.DMA,
    ]
    return pl.kernel(
        functools.partial(_sc_agg_body, with_deg),
        out_type=out_type,
        mesh=mesh,
        scratch_types=scratch,
        name="sc_edge_agg" + ("_deg" if with_deg else ""),
    )


_sc_agg_with_deg = _make_sc_agg(True)
_sc_agg_no_deg = _make_sc_agg(False)


def _tc_conv_body(h_ref, p_ref, degp_ref, ws_ref, wn_ref, b_ref, g_ref,
                  bb_ref, out_ref):
    deg = degp_ref[0, :N, 0:1] + degp_ref[1, :N, 0:1]
    inv = 1.0 / jnp.maximum(deg, 1.0)
    agg = (p_ref[0, :N, :] + p_ref[1, :N, :]) * inv
    y = (jnp.dot(h_ref[...], ws_ref[...], preferred_element_type=jnp.float32)
         + jnp.dot(agg, wn_ref[...], preferred_element_type=jnp.float32)
         + b_ref[...])
    m = jnp.mean(y, axis=0, keepdims=True)
    v = jnp.mean((y - m) ** 2, axis=0, keepdims=True)
    yn = (y - m) * lax.rsqrt(v + EPS) * g_ref[...] + bb_ref[...]
    out_ref[...] = jnp.maximum(yn, 0.0)


def _tc_conv(h, p, degp, ws, wn, b, g, bb):
    return pl.pallas_call(
        _tc_conv_body,
        out_shape=jax.ShapeDtypeStruct((N, D), jnp.float32),
    )(h, p, degp, ws, wn, b, g, bb)


def _tc_rest_body(h_ref, p_ref, degp_ref, ws_ref, wn_ref, b_ref, g_ref,
                  bb_ref, wl0_ref, bl0_ref, lg0_ref, lb0_ref, wl1_ref,
                  bl1_ref, lg1_ref, lb1_ref, wh_ref, bh_ref, mask_ref,
                  out_ref):
    deg = degp_ref[0, :N, 0:1] + degp_ref[1, :N, 0:1]
    inv = 1.0 / jnp.maximum(deg, 1.0)
    agg = (p_ref[0, :N, :] + p_ref[1, :N, :]) * inv
    y = (jnp.dot(h_ref[...], ws_ref[...], preferred_element_type=jnp.float32)
         + jnp.dot(agg, wn_ref[...], preferred_element_type=jnp.float32)
         + b_ref[...])
    m = jnp.mean(y, axis=0, keepdims=True)
    v = jnp.mean((y - m) ** 2, axis=0, keepdims=True)
    z = jnp.maximum((y - m) * lax.rsqrt(v + EPS) * g_ref[...] + bb_ref[...],
                    0.0)

    def mlp(t, wl, bl, lg, lb):
        u = jnp.dot(t, wl, preferred_element_type=jnp.float32) + bl
        mu = jnp.mean(u, axis=1, keepdims=True)
        var = jnp.mean((u - mu) ** 2, axis=1, keepdims=True)
        return jnp.maximum((u - mu) * lax.rsqrt(var + EPS) * lg + lb, 0.0)

    z = mlp(z, wl0_ref[...], bl0_ref[...], lg0_ref[...], lb0_ref[...])
    z = mlp(z, wl1_ref[...], bl1_ref[...], lg1_ref[...], lb1_ref[...])
    o = jnp.dot(z, wh_ref[...], preferred_element_type=jnp.float32) + bh_ref[...]
    out_ref[...] = o * mask_ref[...]


def _tc_rest(h, p, degp, ws, wn, b, g, bb, wl0, bl0, lg0, lb0, wl1, bl1,
             lg1, lb1, wh, bh, mask):
    return pl.pallas_call(
        _tc_rest_body,
        out_shape=jax.ShapeDtypeStruct((N, 1), jnp.float32),
    )(h, p, degp, ws, wn, b, g, bb, wl0, bl0, lg0, lb0, wl1, bl1, lg1, lb1,
      wh, bh, mask)


def kernel(x, edge_index, regression_mask,
           W_self0, W_neigh0, b_conv0, bn_g0, bn_b0,
           W_self1, W_neigh1, b_conv1, bn_g1, bn_b1,
           W_lin0, b_lin0, ln_g0, ln_b0,
           W_lin1, b_lin1, ln_g1, ln_b1,
           W_head, b_head):
    pad = E_PAD - E
    src = jnp.concatenate([edge_index[0], jnp.zeros((pad,), jnp.int32)])
    dst = jnp.concatenate(
        [edge_index[1], jnp.full((pad,), N, jnp.int32)]).reshape(NW, CPT, K)
    z128 = jnp.zeros((N_PAD, D), dtype=jnp.float32)
    maskf = regression_mask.astype(jnp.float32).reshape(N, 1)

    p0, degp = _sc_agg_with_deg(x, src, dst, z128)
    h1 = _tc_conv(x, p0, degp,
                  W_self0, W_neigh0, b_conv0.reshape(1, D),
                  bn_g0.reshape(1, D), bn_b0.reshape(1, D))
    p1, _ = _sc_agg_no_deg(h1, src, dst, z128)
    out = _tc_rest(h1, p1, degp,
                   W_self1, W_neigh1, b_conv1.reshape(1, D),
                   bn_g1.reshape(1, D), bn_b1.reshape(1, D),
                   W_lin0, b_lin0.reshape(1, D), ln_g0.reshape(1, D),
                   ln_b0.reshape(1, D),
                   W_lin1, b_lin1.reshape(1, D), ln_g1.reshape(1, D),
                   ln_b1.reshape(1, D),
                   W_head, b_head.reshape(1, 1), maskf)
    return out[:, 0]


# speedup vs baseline: 2.1933x; 2.1933x over previous
import functools

import jax
import jax.numpy as jnp
from jax import lax
from jax.experimental import pallas as pl
from jax.experimental.pallas import tpu as pltpu
from jax.experimental.pallas import tpu_sc as plsc

N = 10000
D = 128
E = 320000
EPS = 1e-5

NC = 2
NS = 16
NW = NC * NS
EPW = E // NW
K = 80
CHUNKS = EPW // K
TRIPLES = (CHUNKS - 5) // 3
N_PAD = 10112
ROWS_PT = N_PAD // NS

assert EPW % K == 0 and CHUNKS == 3 * TRIPLES + 5
assert N_PAD % (8 * NS) == 0 and N_PAD >= N + 1


def _sc_agg_body(with_deg, h_hbm, src_hbm, dst_hbm, z128_hbm, agg_out,
                 deg_out, agg_sh, si0, di0, rb0, si1, di1, rb1, si2, di2,
                 rb2, sg0, sg1, sg2, ss0, ss1, ss2):
    cid = lax.axis_index("c")
    sid = lax.axis_index("s")
    wid = sid * NC + cid
    r0 = sid * ROWS_PT
    ebase = wid * EPW

    pltpu.sync_copy(z128_hbm.at[pl.ds(r0, ROWS_PT)],
                    agg_sh.at[pl.ds(r0, ROWS_PT)])
    plsc.subcore_barrier()

    def load(c, si, di):
        pltpu.sync_copy(src_hbm.at[pl.ds(ebase + c * K, K)], si)
        pltpu.sync_copy(dst_hbm.at[pl.ds(ebase + c * K, K)], di)

    def fire_g(si, rb, sg):
        return pltpu.async_copy(h_hbm.at[si], rb, sg)

    def fire_s(rb, di, ss):
        return pltpu.async_copy(rb, agg_sh.at[di], ss, add=True)

    load(0, si0, di0)
    gd0 = fire_g(si0, rb0, sg0)
    load(1, si1, di1)
    gd1 = fire_g(si1, rb1, sg1)
    load(2, si2, di2)
    gd2 = fire_g(si2, rb2, sg2)
    gd0.wait()
    sd0 = fire_s(rb0, di0, ss0)
    gd1.wait()
    sd1 = fire_s(rb1, di1, ss1)
    gd2.wait()
    sd2 = fire_s(rb2, di2, ss2)

    def triple(j, carry):
        c = 3 * j
        sd0.wait()
        load(c + 3, si0, di0)
        fire_g(si0, rb0, sg0)
        sd1.wait()
        load(c + 4, si1, di1)
        fire_g(si1, rb1, sg1)
        sd2.wait()
        load(c + 5, si2, di2)
        fire_g(si2, rb2, sg2)
        gd0.wait()
        fire_s(rb0, di0, ss0)
        gd1.wait()
        fire_s(rb1, di1, ss1)
        gd2.wait()
        fire_s(rb2, di2, ss2)
        return carry

    lax.fori_loop(0, TRIPLES, triple, 0)

    sd0.wait()
    load(CHUNKS - 2, si0, di0)
    fire_g(si0, rb0, sg0)
    sd1.wait()
    load(CHUNKS - 1, si1, di1)
    fire_g(si1, rb1, sg1)
    gd0.wait()
    fire_s(rb0, di0, ss0)
    gd1.wait()
    fire_s(rb1, di1, ss1)
    sd2.wait()
    sd0.wait()
    sd1.wait()

    plsc.subcore_barrier()
    pltpu.sync_copy(agg_sh.at[pl.ds(r0, ROWS_PT)],
                    agg_out.at[cid, pl.ds(r0, ROWS_PT)])

    if with_deg:
        plsc.subcore_barrier()
        pltpu.sync_copy(z128_hbm.at[pl.ds(r0, ROWS_PT)],
                        agg_sh.at[pl.ds(r0, ROWS_PT)])
        for i in range(K):
            for j in range(D // 16):
                rb0[i, pl.ds(j * 16, 16)] = jnp.ones((16,), jnp.float32)
        plsc.subcore_barrier()

        def dload(c, di):
            pltpu.sync_copy(dst_hbm.at[pl.ds(ebase + c * K, K)], di)

        dload(0, di0)
        dd0 = fire_s(rb0, di0, ss0)
        dload(1, di1)
        dd1 = fire_s(rb0, di1, ss1)
        dload(2, di2)
        dd2 = fire_s(rb0, di2, ss2)

        def dtriple(j, carry):
            c = 3 * j
            dd0.wait()
            dload(c + 3, di0)
            fire_s(rb0, di0, ss0)
            dd1.wait()
            dload(c + 4, di1)
            fire_s(rb0, di1, ss1)
            dd2.wait()
            dload(c + 5, di2)
            fire_s(rb0, di2, ss2)
            return carry

        lax.fori_loop(0, TRIPLES, dtriple, 0)
        dd0.wait()
        dload(CHUNKS - 2, di0)
        fire_s(rb0, di0, ss0)
        dd1.wait()
        dload(CHUNKS - 1, di1)
        fire_s(rb0, di1, ss1)
        dd2.wait()
        dd0.wait()
        dd1.wait()

        plsc.subcore_barrier()
        pltpu.sync_copy(agg_sh.at[pl.ds(r0, ROWS_PT)],
                        deg_out.at[cid, pl.ds(r0, ROWS_PT)])


def _make_sc_agg(with_deg):
    mesh = plsc.VectorSubcoreMesh(core_axis_name="c", subcore_axis_name="s")
    out_type = (jax.ShapeDtypeStruct((NC, N_PAD, D), jnp.float32),
                jax.ShapeDtypeStruct((NC, N_PAD, D), jnp.float32))
    scratch = [pltpu.VMEM_SHARED((N_PAD, D), jnp.float32)]
    for _ in range(3):
        scratch += [pltpu.VMEM((K,), jnp.int32),
                    pltpu.VMEM((K,), jnp.int32),
                    pltpu.VMEM((K, D), jnp.float32)]
    scratch += [pltpu.SemaphoreType.DMA] * 6
    return pl.kernel(
        functools.partial(_sc_agg_body, with_deg),
        out_type=out_type,
        mesh=mesh,
        scratch_types=scratch,
        name="sc_edge_agg" + ("_deg" if with_deg else ""),
    )


_sc_agg_with_deg = _make_sc_agg(True)
_sc_agg_no_deg = _make_sc_agg(False)


def _tc_conv_body(h_ref, p_ref, degp_ref, ws_ref, wn_ref, b_ref, g_ref,
                  bb_ref, out_ref):
    deg = degp_ref[0, :N, 0:1] + degp_ref[1, :N, 0:1]
    inv = 1.0 / jnp.maximum(deg, 1.0)
    agg = (p_ref[0, :N, :] + p_ref[1, :N, :]) * inv
    y = (jnp.dot(h_ref[...], ws_ref[...], preferred_element_type=jnp.float32)
         + jnp.dot(agg, wn_ref[...], preferred_element_type=jnp.float32)
         + b_ref[...])
    m = jnp.mean(y, axis=0, keepdims=True)
    v = jnp.mean((y - m) ** 2, axis=0, keepdims=True)
    yn = (y - m) * lax.rsqrt(v + EPS) * g_ref[...] + bb_ref[...]
    out_ref[...] = jnp.maximum(yn, 0.0)


def _tc_conv(h, p, degp, ws, wn, b, g, bb):
    return pl.pallas_call(
        _tc_conv_body,
        out_shape=jax.ShapeDtypeStruct((N, D), jnp.float32),
    )(h, p, degp, ws, wn, b, g, bb)


def _tc_rest_body(h_ref, p_ref, degp_ref, ws_ref, wn_ref, b_ref, g_ref,
                  bb_ref, wl0_ref, bl0_ref, lg0_ref, lb0_ref, wl1_ref,
                  bl1_ref, lg1_ref, lb1_ref, wh_ref, bh_ref, mask_ref,
                  out_ref):
    deg = degp_ref[0, :N, 0:1] + degp_ref[1, :N, 0:1]
    inv = 1.0 / jnp.maximum(deg, 1.0)
    agg = (p_ref[0, :N, :] + p_ref[1, :N, :]) * inv
    y = (jnp.dot(h_ref[...], ws_ref[...], preferred_element_type=jnp.float32)
         + jnp.dot(agg, wn_ref[...], preferred_element_type=jnp.float32)
         + b_ref[...])
    m = jnp.mean(y, axis=0, keepdims=True)
    v = jnp.mean((y - m) ** 2, axis=0, keepdims=True)
    z = jnp.maximum((y - m) * lax.rsqrt(v + EPS) * g_ref[...] + bb_ref[...],
                    0.0)

    def mlp(t, wl, bl, lg, lb):
        u = jnp.dot(t, wl, preferred_element_type=jnp.float32) + bl
        mu = jnp.mean(u, axis=1, keepdims=True)
        var = jnp.mean((u - mu) ** 2, axis=1, keepdims=True)
        return jnp.maximum((u - mu) * lax.rsqrt(var + EPS) * lg + lb, 0.0)

    z = mlp(z, wl0_ref[...], bl0_ref[...], lg0_ref[...], lb0_ref[...])
    z = mlp(z, wl1_ref[...], bl1_ref[...], lg1_ref[...], lb1_ref[...])
    o = jnp.dot(z, wh_ref[...], preferred_element_type=jnp.float32) + bh_ref[...]
    out_ref[...] = o * mask_ref[...]


def _tc_rest(h, p, degp, ws, wn, b, g, bb, wl0, bl0, lg0, lb0, wl1, bl1,
             lg1, lb1, wh, bh, mask):
    return pl.pallas_call(
        _tc_rest_body,
        out_shape=jax.ShapeDtypeStruct((N, 1), jnp.float32),
    )(h, p, degp, ws, wn, b, g, bb, wl0, bl0, lg0, lb0, wl1, bl1, lg1, lb1,
      wh, bh, mask)


def kernel(x, edge_index, regression_mask,
           W_self0, W_neigh0, b_conv0, bn_g0, bn_b0,
           W_self1, W_neigh1, b_conv1, bn_g1, bn_b1,
           W_lin0, b_lin0, ln_g0, ln_b0,
           W_lin1, b_lin1, ln_g1, ln_b1,
           W_head, b_head):
    src = edge_index[0]
    dst = edge_index[1]
    z128 = jnp.zeros((N_PAD, D), dtype=jnp.float32)
    maskf = regression_mask.astype(jnp.float32).reshape(N, 1)

    p0, degp = _sc_agg_with_deg(x, src, dst, z128)
    h1 = _tc_conv(x, p0, degp,
                  W_self0, W_neigh0, b_conv0.reshape(1, D),
                  bn_g0.reshape(1, D), bn_b0.reshape(1, D))
    p1, _ = _sc_agg_no_deg(h1, src, dst, z128)
    out = _tc_rest(h1, p1, degp,
                   W_self1, W_neigh1, b_conv1.reshape(1, D),
                   bn_g1.reshape(1, D), bn_b1.reshape(1, D),
                   W_lin0, b_lin0.reshape(1, D), ln_g0.reshape(1, D),
                   ln_b0.reshape(1, D),
                   W_lin1, b_lin1.reshape(1, D), ln_g1.reshape(1, D),
                   ln_b1.reshape(1, D),
                   W_head, b_head.reshape(1, 1), maskf)
    return out[:, 0]


# speedup vs baseline: 2.7820x; 1.2684x over previous
import functools

import jax
import jax.numpy as jnp
from jax import lax
from jax.experimental import pallas as pl
from jax.experimental.pallas import tpu as pltpu
from jax.experimental.pallas import tpu_sc as plsc

N = 10000
D = 128
E = 320000
EPS = 1e-5

NC = 2
NS = 16
NW = NC * NS
EPW = E // NW
K = 80
CHUNKS = EPW // K
TRIPLES = (CHUNKS - 5) // 3
PAIR_ITERS = (CHUNKS - 11) // 6
N_PAD = 10112
ROWS_PT = N_PAD // NS

assert EPW % K == 0 and CHUNKS == 3 * TRIPLES + 5
assert CHUNKS == 6 * PAIR_ITERS + 11
assert N_PAD % (8 * NS) == 0 and N_PAD >= N + 1


def _sc_agg_body(with_deg, h_hbm, src_hbm, dst_hbm, z128_hbm, agg_out,
                 deg_out, agg_sh,
                 siA0, diA0, siA1, diA1, siA2, diA2,
                 siB0, diB0, siB1, diB1, siB2, diB2,
                 rb0, rb1, rb2, semA, semB, sg0, sg1, sg2, ss0, ss1, ss2):
    cid = lax.axis_index("c")
    sid = lax.axis_index("s")
    wid = sid * NC + cid
    r0 = sid * ROWS_PT
    ebase = wid * EPW

    pltpu.sync_copy(z128_hbm.at[pl.ds(r0, ROWS_PT)],
                    agg_sh.at[pl.ds(r0, ROWS_PT)])
    plsc.subcore_barrier()

    def fire_idx(c, si, di, sem):
        d1 = pltpu.async_copy(src_hbm.at[pl.ds(ebase + c * K, K)], si, sem)
        d2 = pltpu.async_copy(dst_hbm.at[pl.ds(ebase + c * K, K)], di, sem)
        return d1, d2

    def fire_g(si, rb, sg):
        return pltpu.async_copy(h_hbm.at[si], rb, sg)

    def fire_s(rb, di, ss):
        return pltpu.async_copy(rb, agg_sh.at[di], ss, add=True)

    A = ((siA0, diA0), (siA1, diA1), (siA2, diA2))
    B = ((siB0, diB0), (siB1, diB1), (siB2, diB2))
    RB = (rb0, rb1, rb2)
    SG = (sg0, sg1, sg2)
    SS = (ss0, ss1, ss2)

    pa = [fire_idx(c, A[c][0], A[c][1], semA) for c in range(3)]
    pb = [fire_idx(3 + c, B[c][0], B[c][1], semB) for c in range(3)]
    gd = [None, None, None]
    sd = [None, None, None]
    for c in range(3):
        pa[c][0].wait()
        pa[c][1].wait()
        gd[c] = fire_g(A[c][0], RB[c], SG[c])
    for c in range(3):
        gd[c].wait()
        sd[c] = fire_s(RB[c], A[c][1], SS[c])

    def half(use, other, other0, n_other=3):
        pw = pa if use is A else pb
        osem = semA if other is A else semB
        for k in range(3):
            pw[k][0].wait()
            pw[k][1].wait()
        for k in range(3):
            sd[k].wait()
            if k < n_other:
                fire_idx(other0 + k, other[k][0], other[k][1], osem)
            gd[k] = fire_g(use[k][0], RB[k], SG[k])
        for k in range(3):
            gd[k].wait()
            sd[k] = fire_s(RB[k], use[k][1], SS[k])

    def pair(m, carry):
        base = 6 * m
        half(B, A, base + 6)
        half(A, B, base + 9)
        return carry

    lax.fori_loop(0, PAIR_ITERS, pair, 0)

    half(B, A, CHUNKS - 5)
    half(A, B, CHUNKS - 2, n_other=2)
    for k in range(2):
        pb[k][0].wait()
        pb[k][1].wait()
    for k in range(2):
        sd[k].wait()
        gd[k] = fire_g(B[k][0], RB[k], SG[k])
    for k in range(2):
        gd[k].wait()
        sd[k] = fire_s(RB[k], B[k][1], SS[k])
    sd[2].wait()
    sd[0].wait()
    sd[1].wait()

    plsc.subcore_barrier()
    pltpu.sync_copy(agg_sh.at[pl.ds(r0, ROWS_PT)],
                    agg_out.at[cid, pl.ds(r0, ROWS_PT)])

    if with_deg:
        plsc.subcore_barrier()
        pltpu.sync_copy(z128_hbm.at[pl.ds(r0, ROWS_PT)],
                        agg_sh.at[pl.ds(r0, ROWS_PT)])
        for i in range(K):
            for j in range(D // 16):
                rb0[i, pl.ds(j * 16, 16)] = jnp.ones((16,), jnp.float32)
        plsc.subcore_barrier()

        def dload(c, di):
            pltpu.sync_copy(dst_hbm.at[pl.ds(ebase + c * K, K)], di)

        dload(0, diA0)
        dd0 = fire_s(rb0, diA0, ss0)
        dload(1, diA1)
        dd1 = fire_s(rb0, diA1, ss1)
        dload(2, diA2)
        dd2 = fire_s(rb0, diA2, ss2)

        def dtriple(j, carry):
            c = 3 * j
            dd0.wait()
            dload(c + 3, diA0)
            fire_s(rb0, diA0, ss0)
            dd1.wait()
            dload(c + 4, diA1)
            fire_s(rb0, diA1, ss1)
            dd2.wait()
            dload(c + 5, diA2)
            fire_s(rb0, diA2, ss2)
            return carry

        lax.fori_loop(0, TRIPLES, dtriple, 0)
        dd0.wait()
        dload(CHUNKS - 2, diA0)
        fire_s(rb0, diA0, ss0)
        dd1.wait()
        dload(CHUNKS - 1, diA1)
        fire_s(rb0, diA1, ss1)
        dd2.wait()
        dd0.wait()
        dd1.wait()

        plsc.subcore_barrier()
        pltpu.sync_copy(agg_sh.at[pl.ds(r0, ROWS_PT)],
                        deg_out.at[cid, pl.ds(r0, ROWS_PT)])


def _make_sc_agg(with_deg):
    mesh = plsc.VectorSubcoreMesh(core_axis_name="c", subcore_axis_name="s")
    out_type = (jax.ShapeDtypeStruct((NC, N_PAD, D), jnp.float32),
                jax.ShapeDtypeStruct((NC, N_PAD, D), jnp.float32))
    scratch = [pltpu.VMEM_SHARED((N_PAD, D), jnp.float32)]
    scratch += [pltpu.VMEM((K,), jnp.int32)] * 12
    scratch += [pltpu.VMEM((K, D), jnp.float32)] * 3
    scratch += [pltpu.SemaphoreType.DMA] * 8
    return pl.kernel(
        functools.partial(_sc_agg_body, with_deg),
        out_type=out_type,
        mesh=mesh,
        scratch_types=scratch,
        name="sc_edge_agg" + ("_deg" if with_deg else ""),
    )


_sc_agg_with_deg = _make_sc_agg(True)
_sc_agg_no_deg = _make_sc_agg(False)


def _tc_conv_body(h_ref, p_ref, degp_ref, ws_ref, wn_ref, b_ref, g_ref,
                  bb_ref, out_ref):
    deg = degp_ref[0, :N, 0:1] + degp_ref[1, :N, 0:1]
    inv = 1.0 / jnp.maximum(deg, 1.0)
    agg = (p_ref[0, :N, :] + p_ref[1, :N, :]) * inv
    y = (jnp.dot(h_ref[...], ws_ref[...], preferred_element_type=jnp.float32)
         + jnp.dot(agg, wn_ref[...], preferred_element_type=jnp.float32)
         + b_ref[...])
    m = jnp.mean(y, axis=0, keepdims=True)
    v = jnp.mean((y - m) ** 2, axis=0, keepdims=True)
    yn = (y - m) * lax.rsqrt(v + EPS) * g_ref[...] + bb_ref[...]
    out_ref[...] = jnp.maximum(yn, 0.0)


def _tc_conv(h, p, degp, ws, wn, b, g, bb):
    return pl.pallas_call(
        _tc_conv_body,
        out_shape=jax.ShapeDtypeStruct((N, D), jnp.float32),
    )(h, p, degp, ws, wn, b, g, bb)


def _tc_rest_body(h_ref, p_ref, degp_ref, ws_ref, wn_ref, b_ref, g_ref,
                  bb_ref, wl0_ref, bl0_ref, lg0_ref, lb0_ref, wl1_ref,
                  bl1_ref, lg1_ref, lb1_ref, wh_ref, bh_ref, mask_ref,
                  out_ref):
    deg = degp_ref[0, :N, 0:1] + degp_ref[1, :N, 0:1]
    inv = 1.0 / jnp.maximum(deg, 1.0)
    agg = (p_ref[0, :N, :] + p_ref[1, :N, :]) * inv
    y = (jnp.dot(h_ref[...], ws_ref[...], preferred_element_type=jnp.float32)
         + jnp.dot(agg, wn_ref[...], preferred_element_type=jnp.float32)
         + b_ref[...])
    m = jnp.mean(y, axis=0, keepdims=True)
    v = jnp.mean((y - m) ** 2, axis=0, keepdims=True)
    z = jnp.maximum((y - m) * lax.rsqrt(v + EPS) * g_ref[...] + bb_ref[...],
                    0.0)

    def mlp(t, wl, bl, lg, lb):
        u = jnp.dot(t, wl, preferred_element_type=jnp.float32) + bl
        mu = jnp.mean(u, axis=1, keepdims=True)
        var = jnp.mean((u - mu) ** 2, axis=1, keepdims=True)
        return jnp.maximum((u - mu) * lax.rsqrt(var + EPS) * lg + lb, 0.0)

    z = mlp(z, wl0_ref[...], bl0_ref[...], lg0_ref[...], lb0_ref[...])
    z = mlp(z, wl1_ref[...], bl1_ref[...], lg1_ref[...], lb1_ref[...])
    o = jnp.dot(z, wh_ref[...], preferred_element_type=jnp.float32) + bh_ref[...]
    out_ref[...] = o * mask_ref[...]


def _tc_rest(h, p, degp, ws, wn, b, g, bb, wl0, bl0, lg0, lb0, wl1, bl1,
             lg1, lb1, wh, bh, mask):
    return pl.pallas_call(
        _tc_rest_body,
        out_shape=jax.ShapeDtypeStruct((N, 1), jnp.float32),
    )(h, p, degp, ws, wn, b, g, bb, wl0, bl0, lg0, lb0, wl1, bl1, lg1, lb1,
      wh, bh, mask)


def kernel(x, edge_index, regression_mask,
           W_self0, W_neigh0, b_conv0, bn_g0, bn_b0,
           W_self1, W_neigh1, b_conv1, bn_g1, bn_b1,
           W_lin0, b_lin0, ln_g0, ln_b0,
           W_lin1, b_lin1, ln_g1, ln_b1,
           W_head, b_head):
    src = edge_index[0]
    dst = edge_index[1]
    z128 = jnp.zeros((N_PAD, D), dtype=jnp.float32)
    maskf = regression_mask.astype(jnp.float32).reshape(N, 1)

    p0, degp = _sc_agg_with_deg(x, src, dst, z128)
    h1 = _tc_conv(x, p0, degp,
                  W_self0, W_neigh0, b_conv0.reshape(1, D),
                  bn_g0.reshape(1, D), bn_b0.reshape(1, D))
    p1, _ = _sc_agg_no_deg(h1, src, dst, z128)
    out = _tc_rest(h1, p1, degp,
                   W_self1, W_neigh1, b_conv1.reshape(1, D),
                   bn_g1.reshape(1, D), bn_b1.reshape(1, D),
                   W_lin0, b_lin0.reshape(1, D), ln_g0.reshape(1, D),
                   ln_b0.reshape(1, D),
                   W_lin1, b_lin1.reshape(1, D), ln_g1.reshape(1, D),
                   ln_b1.reshape(1, D),
                   W_head, b_head.reshape(1, 1), maskf)
    return out[:, 0]


# speedup vs baseline: 2.7934x; 1.0041x over previous
import functools

import jax
import jax.numpy as jnp
from jax import lax
from jax.experimental import pallas as pl
from jax.experimental.pallas import tpu as pltpu
from jax.experimental.pallas import tpu_sc as plsc

N = 10000
D = 128
E = 320000
EPS = 1e-5

NC = 2
NS = 16
NW = NC * NS
EPW = E // NW
K = 80
CHUNKS = EPW // K
TRIPLES = (CHUNKS - 5) // 3
PAIR_ITERS = (CHUNKS - 11) // 6
N_PAD = 10112
ROWS_PT = N_PAD // NS

assert EPW % K == 0 and CHUNKS == 3 * TRIPLES + 5
assert CHUNKS == 6 * PAIR_ITERS + 11
assert N_PAD % (8 * NS) == 0 and N_PAD >= N + 1


def _sc_agg_body(with_deg, h_hbm, src_hbm, dst_hbm, z128_hbm, agg_out,
                 deg_out, agg_sh,
                 siA0, diA0, siA1, diA1, siA2, diA2,
                 siB0, diB0, siB1, diB1, siB2, diB2,
                 rb0, rb1, rb2, semA, semB, sg0, sg1, sg2, ss0, ss1, ss2):
    cid = lax.axis_index("c")
    sid = lax.axis_index("s")
    wid = sid * NC + cid
    r0 = sid * ROWS_PT
    ebase = wid * EPW

    pltpu.sync_copy(z128_hbm.at[pl.ds(r0, ROWS_PT)],
                    agg_sh.at[pl.ds(r0, ROWS_PT)])
    plsc.subcore_barrier()

    def fire_idx(c, si, di, sem):
        d1 = pltpu.async_copy(src_hbm.at[pl.ds(ebase + c * K, K)], si, sem)
        d2 = pltpu.async_copy(dst_hbm.at[pl.ds(ebase + c * K, K)], di, sem)
        return d1, d2

    def fire_g(si, rb, sg):
        return pltpu.async_copy(h_hbm.at[si], rb, sg)

    def fire_s(rb, di, ss):
        return pltpu.async_copy(rb, agg_sh.at[di], ss, add=True)

    A = ((siA0, diA0), (siA1, diA1), (siA2, diA2))
    B = ((siB0, diB0), (siB1, diB1), (siB2, diB2))
    RB = (rb0, rb1, rb2)
    SG = (sg0, sg1, sg2)
    SS = (ss0, ss1, ss2)

    pa = [fire_idx(c, A[c][0], A[c][1], semA) for c in range(3)]
    pb = [fire_idx(3 + c, B[c][0], B[c][1], semB) for c in range(3)]
    gd = [None, None, None]
    sd = [None, None, None]
    for c in range(3):
        pa[c][0].wait()
        pa[c][1].wait()
        gd[c] = fire_g(A[c][0], RB[c], SG[c])
    for c in range(3):
        gd[c].wait()
        sd[c] = fire_s(RB[c], A[c][1], SS[c])

    def half(use, other, other0, n_other=3):
        pw = pa if use is A else pb
        osem = semA if other is A else semB
        for k in range(3):
            pw[k][0].wait()
            pw[k][1].wait()
        for k in range(3):
            sd[k].wait()
            if k < n_other:
                fire_idx(other0 + k, other[k][0], other[k][1], osem)
            gd[k] = fire_g(use[k][0], RB[k], SG[k])
        for k in range(3):
            gd[k].wait()
            sd[k] = fire_s(RB[k], use[k][1], SS[k])

    def pair(m, carry):
        base = 6 * m
        half(B, A, base + 6)
        half(A, B, base + 9)
        return carry

    lax.fori_loop(0, PAIR_ITERS, pair, 0)

    half(B, A, CHUNKS - 5)
    half(A, B, CHUNKS - 2, n_other=2)
    for k in range(2):
        pb[k][0].wait()
        pb[k][1].wait()
    for k in range(2):
        sd[k].wait()
        gd[k] = fire_g(B[k][0], RB[k], SG[k])
    for k in range(2):
        gd[k].wait()
        sd[k] = fire_s(RB[k], B[k][1], SS[k])
    sd[2].wait()
    sd[0].wait()
    sd[1].wait()

    plsc.subcore_barrier()
    pltpu.sync_copy(agg_sh.at[pl.ds(r0, ROWS_PT)],
                    agg_out.at[cid, pl.ds(r0, ROWS_PT)])

    if with_deg:
        plsc.subcore_barrier()
        pltpu.sync_copy(z128_hbm.at[pl.ds(r0, ROWS_PT)],
                        agg_sh.at[pl.ds(r0, ROWS_PT)])
        for i in range(K):
            for j in range(D // 16):
                rb0[i, pl.ds(j * 16, 16)] = jnp.ones((16,), jnp.float32)
        plsc.subcore_barrier()

        DA = (diA0, diA1, diA2)
        DB = (diB0, diB1, diB2)

        def dfire_idx(c, di, sem):
            return pltpu.async_copy(dst_hbm.at[pl.ds(ebase + c * K, K)],
                                    di, sem)

        dpa = [dfire_idx(c, DA[c], semA) for c in range(3)]
        dpb = [dfire_idx(3 + c, DB[c], semB) for c in range(3)]
        dsd = [None, None, None]
        for k in range(3):
            dpa[k].wait()
            dsd[k] = fire_s(rb0, DA[k], SS[k])

        def dhalf(use, pw, other, other0, osem, n_other=3):
            for k in range(3):
                pw[k].wait()
            for k in range(3):
                dsd[k].wait()
                if k < n_other:
                    dfire_idx(other0 + k, other[k], osem)
                dsd[k] = fire_s(rb0, use[k], SS[k])

        def dpair(m, carry):
            base = 6 * m
            dhalf(DB, dpb, DA, base + 6, semA)
            dhalf(DA, dpa, DB, base + 9, semB)
            return carry

        lax.fori_loop(0, PAIR_ITERS, dpair, 0)
        dhalf(DB, dpb, DA, CHUNKS - 5, semA)
        dhalf(DA, dpa, DB, CHUNKS - 2, semB, n_other=2)
        for k in range(2):
            dpb[k].wait()
            dsd[k].wait()
            dsd[k] = fire_s(rb0, DB[k], SS[k])
        dsd[2].wait()
        dsd[0].wait()
        dsd[1].wait()

        plsc.subcore_barrier()
        pltpu.sync_copy(agg_sh.at[pl.ds(r0, ROWS_PT)],
                        deg_out.at[cid, pl.ds(r0, ROWS_PT)])


def _make_sc_agg(with_deg):
    mesh = plsc.VectorSubcoreMesh(core_axis_name="c", subcore_axis_name="s")
    out_type = (jax.ShapeDtypeStruct((NC, N_PAD, D), jnp.float32),
                jax.ShapeDtypeStruct((NC, N_PAD, D), jnp.float32))
    scratch = [pltpu.VMEM_SHARED((N_PAD, D), jnp.float32)]
    scratch += [pltpu.VMEM((K,), jnp.int32)] * 12
    scratch += [pltpu.VMEM((K, D), jnp.float32)] * 3
    scratch += [pltpu.SemaphoreType.DMA] * 8
    return pl.kernel(
        functools.partial(_sc_agg_body, with_deg),
        out_type=out_type,
        mesh=mesh,
        scratch_types=scratch,
        name="sc_edge_agg" + ("_deg" if with_deg else ""),
    )


_sc_agg_with_deg = _make_sc_agg(True)
_sc_agg_no_deg = _make_sc_agg(False)


def _tc_conv_body(h_ref, p_ref, degp_ref, ws_ref, wn_ref, b_ref, g_ref,
                  bb_ref, out_ref):
    deg = degp_ref[0, :N, 0:1] + degp_ref[1, :N, 0:1]
    inv = 1.0 / jnp.maximum(deg, 1.0)
    agg = (p_ref[0, :N, :] + p_ref[1, :N, :]) * inv
    y = (jnp.dot(h_ref[...], ws_ref[...], preferred_element_type=jnp.float32)
         + jnp.dot(agg, wn_ref[...], preferred_element_type=jnp.float32)
         + b_ref[...])
    m = jnp.mean(y, axis=0, keepdims=True)
    v = jnp.mean((y - m) ** 2, axis=0, keepdims=True)
    yn = (y - m) * lax.rsqrt(v + EPS) * g_ref[...] + bb_ref[...]
    out_ref[...] = jnp.maximum(yn, 0.0)


def _tc_conv(h, p, degp, ws, wn, b, g, bb):
    return pl.pallas_call(
        _tc_conv_body,
        out_shape=jax.ShapeDtypeStruct((N, D), jnp.float32),
    )(h, p, degp, ws, wn, b, g, bb)


def _tc_rest_body(h_ref, p_ref, degp_ref, ws_ref, wn_ref, b_ref, g_ref,
                  bb_ref, wl0_ref, bl0_ref, lg0_ref, lb0_ref, wl1_ref,
                  bl1_ref, lg1_ref, lb1_ref, wh_ref, bh_ref, mask_ref,
                  out_ref):
    deg = degp_ref[0, :N, 0:1] + degp_ref[1, :N, 0:1]
    inv = 1.0 / jnp.maximum(deg, 1.0)
    agg = (p_ref[0, :N, :] + p_ref[1, :N, :]) * inv
    y = (jnp.dot(h_ref[...], ws_ref[...], preferred_element_type=jnp.float32)
         + jnp.dot(agg, wn_ref[...], preferred_element_type=jnp.float32)
         + b_ref[...])
    m = jnp.mean(y, axis=0, keepdims=True)
    v = jnp.mean((y - m) ** 2, axis=0, keepdims=True)
    z = jnp.maximum((y - m) * lax.rsqrt(v + EPS) * g_ref[...] + bb_ref[...],
                    0.0)

    def mlp(t, wl, bl, lg, lb):
        u = jnp.dot(t, wl, preferred_element_type=jnp.float32) + bl
        mu = jnp.mean(u, axis=1, keepdims=True)
        var = jnp.mean((u - mu) ** 2, axis=1, keepdims=True)
        return jnp.maximum((u - mu) * lax.rsqrt(var + EPS) * lg + lb, 0.0)

    z = mlp(z, wl0_ref[...], bl0_ref[...], lg0_ref[...], lb0_ref[...])
    z = mlp(z, wl1_ref[...], bl1_ref[...], lg1_ref[...], lb1_ref[...])
    o = jnp.dot(z, wh_ref[...], preferred_element_type=jnp.float32) + bh_ref[...]
    out_ref[...] = o * mask_ref[...]


def _tc_rest(h, p, degp, ws, wn, b, g, bb, wl0, bl0, lg0, lb0, wl1, bl1,
             lg1, lb1, wh, bh, mask):
    return pl.pallas_call(
        _tc_rest_body,
        out_shape=jax.ShapeDtypeStruct((N, 1), jnp.float32),
    )(h, p, degp, ws, wn, b, g, bb, wl0, bl0, lg0, lb0, wl1, bl1, lg1, lb1,
      wh, bh, mask)


def kernel(x, edge_index, regression_mask,
           W_self0, W_neigh0, b_conv0, bn_g0, bn_b0,
           W_self1, W_neigh1, b_conv1, bn_g1, bn_b1,
           W_lin0, b_lin0, ln_g0, ln_b0,
           W_lin1, b_lin1, ln_g1, ln_b1,
           W_head, b_head):
    src = edge_index[0]
    dst = edge_index[1]
    z128 = jnp.zeros((N_PAD, D), dtype=jnp.float32)
    maskf = regression_mask.astype(jnp.float32).reshape(N, 1)

    p0, degp = _sc_agg_with_deg(x, src, dst, z128)
    h1 = _tc_conv(x, p0, degp,
                  W_self0, W_neigh0, b_conv0.reshape(1, D),
                  bn_g0.reshape(1, D), bn_b0.reshape(1, D))
    p1, _ = _sc_agg_no_deg(h1, src, dst, z128)
    out = _tc_rest(h1, p1, degp,
                   W_self1, W_neigh1, b_conv1.reshape(1, D),
                   bn_g1.reshape(1, D), bn_b1.reshape(1, D),
                   W_lin0, b_lin0.reshape(1, D), ln_g0.reshape(1, D),
                   ln_b0.reshape(1, D),
                   W_lin1, b_lin1.reshape(1, D), ln_g1.reshape(1, D),
                   ln_b1.reshape(1, D),
                   W_head, b_head.reshape(1, 1), maskf)
    return out[:, 0]
